# baseline (speedup 1.0000x reference)
import jax
import jax.numpy as jnp
from jax.experimental import pallas as pl
import jax.experimental.pallas.tpu as pltpu

N = 10000
D = 128
BM = 240
BKH = 5120
KPAD = 2 * BKH
KVAL2 = N - BKH


def _dot(a, b):
    return jax.lax.dot_general(a, b, (((1,), (0,)), ((), ())),
                               preferred_element_type=jnp.float32)


def _fused_kernel(al0_ref, al1_ref, ah0_ref, ah1_ref, x_ref, xb_ref,
                  wl_ref, wh_ref, wm_ref, avl_ref, avh_ref, avm_ref,
                  att_ref, out_ref, u_s, v_s):
    i = pl.program_id(0)

    @pl.when(i == 0)
    def _init():
        xb = x_ref[...]
        u_s[pl.ds(0, N), :] = _dot(xb, wl_ref[...])
        v_s[pl.ds(0, N), :] = _dot(xb, wh_ref[...])
        u_s[pl.ds(N, KPAD - N), :] = jnp.zeros((KPAD - N, D), jnp.float32)
        v_s[pl.ds(N, KPAD - N), :] = jnp.zeros((KPAD - N, D), jnp.float32)

    col = jax.lax.broadcasted_iota(jnp.int32, (BM, BKH), 1)
    valid = col < KVAL2
    al1 = jnp.where(valid, al1_ref[...], 0.0)
    ah1 = jnp.where(valid, ah1_ref[...], 0.0)

    ol = jnp.maximum(_dot(al0_ref[...], u_s[pl.ds(0, BKH), :])
                     + _dot(al1, u_s[pl.ds(BKH, BKH), :]), 0.0)
    oh = jnp.maximum(_dot(ah0_ref[...], v_s[pl.ds(0, BKH), :])
                     + _dot(ah1, v_s[pl.ds(BKH, BKH), :]), 0.0)
    m = jnp.maximum(_dot(xb_ref[...], wm_ref[...]), 0.0)
    ll = _dot(ol, avl_ref[...])
    lh = _dot(oh, avh_ref[...])
    lm = _dot(m, avm_ref[...])
    logits = jnp.concatenate([ll, lh, lm], axis=1)
    z = _dot(jax.nn.sigmoid(logits), att_ref[...]) * (1.0 / 3.0)
    zmax = jnp.max(z, axis=1, keepdims=True)
    e = jnp.exp(z - zmax)
    att = e / jnp.sum(e, axis=1, keepdims=True)
    out_ref[...] = 3.0 * (att[:, 0:1] * ol + att[:, 1:2] * oh + att[:, 2:3] * m)


@jax.jit
def kernel(input, adj_low, adj_high, weight_low, weight_high, weight_mlp,
           att_vec_low, att_vec_high, att_vec_mlp, att_vec):
    nb = -(-N // BM)
    out = pl.pallas_call(
        _fused_kernel,
        grid=(nb,),
        in_specs=[
            pl.BlockSpec((BM, BKH), lambda i: (i, 0)),
            pl.BlockSpec((BM, BKH), lambda i: (i, 1)),
            pl.BlockSpec((BM, BKH), lambda i: (i, 0)),
            pl.BlockSpec((BM, BKH), lambda i: (i, 1)),
            pl.BlockSpec((N, D), lambda i: (0, 0)),
            pl.BlockSpec((BM, D), lambda i: (i, 0)),
            pl.BlockSpec((D, D), lambda i: (0, 0)),
            pl.BlockSpec((D, D), lambda i: (0, 0)),
            pl.BlockSpec((D, D), lambda i: (0, 0)),
            pl.BlockSpec((D, 1), lambda i: (0, 0)),
            pl.BlockSpec((D, 1), lambda i: (0, 0)),
            pl.BlockSpec((D, 1), lambda i: (0, 0)),
            pl.BlockSpec((3, 3), lambda i: (0, 0)),
        ],
        out_specs=pl.BlockSpec((BM, D), lambda i: (i, 0)),
        out_shape=jax.ShapeDtypeStruct((N, D), jnp.float32),
        scratch_shapes=[
            pltpu.VMEM((KPAD, D), jnp.float32),
            pltpu.VMEM((KPAD, D), jnp.float32),
        ],
    )(adj_low, adj_low, adj_high, adj_high, input, input,
      weight_low, weight_high, weight_mlp,
      att_vec_low, att_vec_high, att_vec_mlp, att_vec)
    return out


# speedup vs baseline: 1.0071x; 1.0071x over previous
import jax
import jax.numpy as jnp
from jax.experimental import pallas as pl
import jax.experimental.pallas.tpu as pltpu

N = 10000
D = 128
BM = 200
BKH = 5120
KPAD = 2 * BKH
KVAL2 = N - BKH


def _dot(a, b):
    return jax.lax.dot_general(a, b, (((1,), (0,)), ((), ())),
                               preferred_element_type=jnp.float32)


def _fused_kernel(al0_ref, al1_ref, ah0_ref, ah1_ref, x_ref,
                  wl_ref, wh_ref, wm_ref, avl_ref, avh_ref, avm_ref,
                  att_ref, out_ref, u_s, v_s):
    i = pl.program_id(0)

    @pl.when(i == 0)
    def _init():
        xb = x_ref[...]
        u_s[pl.ds(0, N), :] = _dot(xb, wl_ref[...])
        v_s[pl.ds(0, N), :] = _dot(xb, wh_ref[...])
        u_s[pl.ds(N, KPAD - N), :] = jnp.zeros((KPAD - N, D), jnp.float32)
        v_s[pl.ds(N, KPAD - N), :] = jnp.zeros((KPAD - N, D), jnp.float32)

    col = jax.lax.broadcasted_iota(jnp.int32, (BM, BKH), 1)
    valid = col < KVAL2
    al1 = jnp.where(valid, al1_ref[...], 0.0)
    ah1 = jnp.where(valid, ah1_ref[...], 0.0)

    ol = jnp.maximum(_dot(al0_ref[...], u_s[pl.ds(0, BKH), :])
                     + _dot(al1, u_s[pl.ds(BKH, BKH), :]), 0.0)
    oh = jnp.maximum(_dot(ah0_ref[...], v_s[pl.ds(0, BKH), :])
                     + _dot(ah1, v_s[pl.ds(BKH, BKH), :]), 0.0)
    x_blk = x_ref[pl.ds(i * BM, BM), :]
    m = jnp.maximum(_dot(x_blk, wm_ref[...]), 0.0)
    ll = _dot(ol, avl_ref[...])
    lh = _dot(oh, avh_ref[...])
    lm = _dot(m, avm_ref[...])
    logits = jnp.concatenate([ll, lh, lm], axis=1)
    z = _dot(jax.nn.sigmoid(logits), att_ref[...]) * (1.0 / 3.0)
    zmax = jnp.max(z, axis=1, keepdims=True)
    e = jnp.exp(z - zmax)
    att = e / jnp.sum(e, axis=1, keepdims=True)
    out_ref[...] = 3.0 * (att[:, 0:1] * ol + att[:, 1:2] * oh + att[:, 2:3] * m)


@jax.jit
def kernel(input, adj_low, adj_high, weight_low, weight_high, weight_mlp,
           att_vec_low, att_vec_high, att_vec_mlp, att_vec):
    nb = N // BM
    out = pl.pallas_call(
        _fused_kernel,
        grid=(nb,),
        in_specs=[
            pl.BlockSpec((BM, BKH), lambda i: (i, 0)),
            pl.BlockSpec((BM, BKH), lambda i: (i, 1)),
            pl.BlockSpec((BM, BKH), lambda i: (i, 0)),
            pl.BlockSpec((BM, BKH), lambda i: (i, 1)),
            pl.BlockSpec((N, D), lambda i: (0, 0)),
            pl.BlockSpec((D, D), lambda i: (0, 0)),
            pl.BlockSpec((D, D), lambda i: (0, 0)),
            pl.BlockSpec((D, D), lambda i: (0, 0)),
            pl.BlockSpec((D, 1), lambda i: (0, 0)),
            pl.BlockSpec((D, 1), lambda i: (0, 0)),
            pl.BlockSpec((D, 1), lambda i: (0, 0)),
            pl.BlockSpec((3, 3), lambda i: (0, 0)),
        ],
        out_specs=pl.BlockSpec((BM, D), lambda i: (i, 0)),
        out_shape=jax.ShapeDtypeStruct((N, D), jnp.float32),
        scratch_shapes=[
            pltpu.VMEM((KPAD, D), jnp.float32),
            pltpu.VMEM((KPAD, D), jnp.float32),
        ],
    )(adj_low, adj_low, adj_high, adj_high, input,
      weight_low, weight_high, weight_mlp,
      att_vec_low, att_vec_high, att_vec_mlp, att_vec)
    return out
